# Initial kernel scaffold; baseline (speedup 1.0000x reference)
import jax
import jax.numpy as jnp
from jax.experimental import pallas as pl


def kernel(x, table, W0, b0, W1, b1, W2, b2, Wout, bout):
    raise NotImplementedError("write your pallas kernel here")


# speedup vs baseline: 4.5658x; 4.5658x over previous
import functools

import jax
import jax.numpy as jnp
from jax import lax
from jax.experimental import pallas as pl
from jax.experimental.pallas import tpu as pltpu
from jax.experimental.pallas import tpu_sc as plsc

N_FEATURES = 26
N_CATEGORIES = 1000
EMBED_DIM = 64
HIDDEN = 512
BATCH = 4096
IN_DIM = N_FEATURES * EMBED_DIM
TOTAL_ROWS = BATCH * N_FEATURES

_NUM_CORES = 2
_NUM_SUBCORES = 16
_NW = _NUM_CORES * _NUM_SUBCORES
_CHUNK = 128
_ROWS_PER_W = TOTAL_ROWS // _NW
_CHUNKS_PER_W = _ROWS_PER_W // _CHUNK

def _sc_gather_body(idx_hbm, table_hbm, out_hbm, idx_v, buf0, buf1, sem0, sem1):
    wid = lax.axis_index("s") * _NUM_CORES + lax.axis_index("c")
    base = wid * _CHUNKS_PER_W
    pltpu.sync_copy(idx_hbm.at[wid], idx_v)

    pltpu.async_copy(table_hbm.at[idx_v.at[0]], buf0, sem0)

    def step(c, _):
        even = lax.rem(c, 2) == 0
        cur_buf, cur_sem = buf0, sem0
        nxt_buf, nxt_sem = buf1, sem1

        def do(cur_buf, cur_sem, nxt_buf, nxt_sem):
            @pl.when(c + 1 < _CHUNKS_PER_W)
            def _():
                pltpu.async_copy(table_hbm.at[idx_v.at[c + 1]], nxt_buf, nxt_sem)
            pltpu.make_async_copy(table_hbm.at[idx_v.at[c]], cur_buf, cur_sem).wait()
            off = pl.multiple_of((base + c) * _CHUNK, _CHUNK)
            pltpu.sync_copy(cur_buf, out_hbm.at[pl.ds(off, _CHUNK)])

        @pl.when(even)
        def _():
            do(buf0, sem0, buf1, sem1)

        @pl.when(jnp.logical_not(even))
        def _():
            do(buf1, sem1, buf0, sem0)

        return 0

    lax.fori_loop(0, _CHUNKS_PER_W, step, 0)


@functools.lru_cache(maxsize=1)
def _sc_gather_fn():
    mesh = plsc.VectorSubcoreMesh(
        core_axis_name="c", subcore_axis_name="s",
        num_cores=_NUM_CORES, num_subcores=_NUM_SUBCORES,
    )
    return pl.kernel(
        _sc_gather_body,
        out_type=jax.ShapeDtypeStruct((TOTAL_ROWS, EMBED_DIM), jnp.float32),
        mesh=mesh,
        scratch_types=[
            pltpu.VMEM((_CHUNKS_PER_W, _CHUNK), jnp.int32),
            pltpu.VMEM((_CHUNK, EMBED_DIM), jnp.float32),
            pltpu.VMEM((_CHUNK, EMBED_DIM), jnp.float32),
            pltpu.SemaphoreType.DMA,
            pltpu.SemaphoreType.DMA,
        ],
        compiler_params=pltpu.CompilerParams(use_tc_tiling_on_sc=False),
    )


_BT = 512


def _mlp_body(h_ref, w0_ref, b0_ref, w1_ref, b1_ref, w2_ref, b2_ref,
              wout_ref, bout_ref, out_ref):
    h = h_ref[...]
    a = jnp.dot(h, w0_ref[...], preferred_element_type=jnp.float32)
    a = jnp.maximum(a + b0_ref[...], 0.0)
    a = jnp.dot(a, w1_ref[...], preferred_element_type=jnp.float32)
    a = jnp.maximum(a + b1_ref[...], 0.0)
    a = jnp.dot(a, w2_ref[...], preferred_element_type=jnp.float32)
    a = jnp.maximum(a + b2_ref[...], 0.0)
    z = jnp.sum(a * wout_ref[...], axis=1, keepdims=True) + bout_ref[...]
    out_ref[...] = 1.0 / (1.0 + jnp.exp(-z))


def _mlp(h, W0, b0, W1, b1, W2, b2, WoutT, bout):
    grid = (BATCH // _BT,)
    return pl.pallas_call(
        _mlp_body,
        grid=grid,
        in_specs=[
            pl.BlockSpec((_BT, IN_DIM), lambda i: (i, 0)),
            pl.BlockSpec((IN_DIM, HIDDEN), lambda i: (0, 0)),
            pl.BlockSpec((1, HIDDEN), lambda i: (0, 0)),
            pl.BlockSpec((HIDDEN, HIDDEN), lambda i: (0, 0)),
            pl.BlockSpec((1, HIDDEN), lambda i: (0, 0)),
            pl.BlockSpec((HIDDEN, HIDDEN), lambda i: (0, 0)),
            pl.BlockSpec((1, HIDDEN), lambda i: (0, 0)),
            pl.BlockSpec((1, HIDDEN), lambda i: (0, 0)),
            pl.BlockSpec((1, 1), lambda i: (0, 0)),
        ],
        out_specs=pl.BlockSpec((_BT, 1), lambda i: (i, 0)),
        out_shape=jax.ShapeDtypeStruct((BATCH, 1), jnp.float32),
    )(h, W0, b0, W1, b1, W2, b2, WoutT, bout)


def kernel(x, table, W0, b0, W1, b1, W2, b2, Wout, bout):
    offsets = (jnp.arange(N_FEATURES, dtype=jnp.int32) * N_CATEGORIES)
    idx = x.astype(jnp.int32) + offsets[None, :]
    idx2 = idx.reshape(_NW, _CHUNKS_PER_W, _CHUNK)
    emb = _sc_gather_fn()(idx2, table)
    h = emb.reshape(BATCH, IN_DIM)
    out = _mlp(
        h,
        W0, b0.reshape(1, HIDDEN),
        W1, b1.reshape(1, HIDDEN),
        W2, b2.reshape(1, HIDDEN),
        Wout.reshape(1, HIDDEN), bout.reshape(1, 1),
    )
    return out
